# baseline (speedup 1.0000x reference)
import functools

import jax
import jax.numpy as jnp
from jax import lax
from jax.experimental import pallas as pl
from jax.experimental.pallas import tpu as pltpu
from jax.experimental.pallas import tpu_sc as plsc

_EMB = 64
_WIN = 5
_HID = 512
_OUT = 50
_B = 16384

_NFLAT = _B * _WIN
_NW = 32
_PER_W = _NFLAT // _NW
_CB = 128
_NCHUNK = _PER_W // _CB

_sc_mesh = plsc.VectorSubcoreMesh(core_axis_name="c", subcore_axis_name="s")


def _repack_body(x_ref, o_ref):
    g = x_ref.shape[0]
    lo = x_ref[pl.Slice(0, g // 2, 2), :]
    hi = x_ref[pl.Slice(1, g // 2, 2), :]
    o_ref[...] = jnp.concatenate([lo, hi], axis=1)


def _make_repack(rows, block):
    return pl.pallas_call(
        _repack_body,
        grid=(rows // block,),
        in_specs=[pl.BlockSpec((block, _EMB), lambda i: (i, 0))],
        out_specs=pl.BlockSpec((block // 2, 2 * _EMB), lambda i: (i, 0)),
        out_shape=jax.ShapeDtypeStruct((rows // 2, 2 * _EMB), jnp.float32),
    )


_repack_word = _make_repack(1000000, 10000)
_repack_affix = _make_repack(100000, 10000)


@functools.partial(
    pl.kernel,
    mesh=_sc_mesh,
    compiler_params=pltpu.CompilerParams(needs_layout_passes=False),
    out_type=jax.ShapeDtypeStruct((_WIN * _EMB, _B), jnp.float32),
    scratch_types=[
        pltpu.VMEM((_PER_W,), jnp.int32),
        pltpu.VMEM((_PER_W,), jnp.int32),
        pltpu.VMEM((_PER_W,), jnp.int32),
        pltpu.VMEM((_PER_W,), jnp.int32),
        pltpu.VMEM((_PER_W,), jnp.int32),
        pltpu.VMEM((_PER_W,), jnp.int32),
        pltpu.VMEM((2, _CB, 2 * _EMB), jnp.float32),
        pltpu.VMEM((2, _CB, 2 * _EMB), jnp.float32),
        pltpu.VMEM((2, _CB, 2 * _EMB), jnp.float32),
        pltpu.VMEM((_EMB, _CB), jnp.float32),
        pltpu.SemaphoreType.DMA,
        pltpu.SemaphoreType.DMA,
        pltpu.SemaphoreType.DMA,
        pltpu.SemaphoreType.DMA,
        pltpu.SemaphoreType.DMA,
        pltpu.SemaphoreType.DMA,
    ],
)
def _sc_gather(w2, p2, s2, qw, qp, qs, parw, parp, pars, out,
               qw_v, qp_v, qs_v, pw_v, pp_v, ps_v,
               rw, rp, rs, comb_t,
               sw0, sw1, sp0, sp1, ss0, ss1):
    wid = lax.axis_index("s") * 2 + lax.axis_index("c")
    base = wid * _PER_W
    pltpu.sync_copy(qw.at[pl.ds(base, _PER_W)], qw_v)
    pltpu.sync_copy(qp.at[pl.ds(base, _PER_W)], qp_v)
    pltpu.sync_copy(qs.at[pl.ds(base, _PER_W)], qs_v)
    pltpu.sync_copy(parw.at[pl.ds(base, _PER_W)], pw_v)
    pltpu.sync_copy(parp.at[pl.ds(base, _PER_W)], pp_v)
    pltpu.sync_copy(pars.at[pl.ds(base, _PER_W)], ps_v)
    sems_w = (sw0, sw1)
    sems_p = (sp0, sp1)
    sems_s = (ss0, ss1)

    def start(c):
        buf = c % 2
        off = pl.ds(c * _CB, _CB)
        return (
            pltpu.async_copy(w2.at[qw_v.at[off]], rw.at[buf], sems_w[buf]),
            pltpu.async_copy(p2.at[qp_v.at[off]], rp.at[buf], sems_p[buf]),
            pltpu.async_copy(s2.at[qs_v.at[off]], rs.at[buf], sems_s[buf]),
        )

    iota = lax.iota(jnp.int32, 16)
    pending = start(0)
    for c in range(_NCHUNK):
        cur = pending
        if c + 1 < _NCHUNK:
            pending = start(c + 1)
        for h in cur:
            h.wait()
        buf = c % 2

        def grp_body(g, _):
            rows16 = g * 16 + iota
            cw = pw_v[pl.ds(c * _CB + g * 16, 16)] * _EMB
            cp = pp_v[pl.ds(c * _CB + g * 16, 16)] * _EMB
            cs = ps_v[pl.ds(c * _CB + g * 16, 16)] * _EMB

            def col_body(e, _):
                v = (plsc.load_gather(rw.at[buf], [rows16, cw + e])
                     + plsc.load_gather(rp.at[buf], [rows16, cp + e])
                     + plsc.load_gather(rs.at[buf], [rows16, cs + e]))
                comb_t[e, pl.ds(g * 16, 16)] = v
                return 0

            lax.fori_loop(0, _EMB, col_body, 0)
            return 0

        lax.fori_loop(0, _CB // 16, grp_body, 0)
        flat = base + c * _CB
        pltpu.sync_copy(
            comb_t,
            out.at[pl.ds((flat // _B) * _EMB, _EMB),
                   pl.ds(flat % _B, _CB)])


def _mlp_body(xt_ref, w1_ref, b1_ref, w2_ref, b2_ref, o_ref):
    ht = jnp.tanh(
        lax.dot_general(w1_ref[...], xt_ref[...],
                        (((0,), (0,)), ((), ())),
                        preferred_element_type=jnp.float32)
        + b1_ref[...])
    o_ref[...] = (
        lax.dot_general(ht, w2_ref[...],
                        (((0,), (0,)), ((), ())),
                        preferred_element_type=jnp.float32)
        + b2_ref[...])


_BM = 2048
_NB = _B // _BM

_mlp = pl.pallas_call(
    _mlp_body,
    grid=(_NB,),
    in_specs=[
        pl.BlockSpec((_WIN * _EMB, _BM), lambda i: (0, i)),
        pl.BlockSpec((_WIN * _EMB, _HID), lambda i: (0, 0)),
        pl.BlockSpec((_HID, 1), lambda i: (0, 0)),
        pl.BlockSpec((_HID, _OUT), lambda i: (0, 0)),
        pl.BlockSpec((1, _OUT), lambda i: (0, 0)),
    ],
    out_specs=pl.BlockSpec((_BM, _OUT), lambda i: (i, 0)),
    out_shape=jax.ShapeDtypeStruct((_B, _OUT), jnp.float32),
)


def kernel(words, prefixes, suffixes, word_emb, prefix_emb, suffix_emb,
           W1, b1, W2, b2):
    w2 = _repack_word(word_emb)
    p2 = _repack_affix(prefix_emb)
    s2 = _repack_affix(suffix_emb)
    iw = words.T.reshape(-1)
    ip = prefixes.T.reshape(-1)
    is_ = suffixes.T.reshape(-1)
    xt = _sc_gather(w2, p2, s2,
                    iw >> 1, ip >> 1, is_ >> 1,
                    iw & 1, ip & 1, is_ & 1)
    return _mlp(xt, W1, b1.reshape(_HID, 1), W2, b2.reshape(1, _OUT))


# speedup vs baseline: 1.5029x; 1.5029x over previous
import functools

import jax
import jax.numpy as jnp
from jax import lax
from jax.experimental import pallas as pl
from jax.experimental.pallas import tpu as pltpu
from jax.experimental.pallas import tpu_sc as plsc

_EMB = 64
_WIN = 5
_HID = 512
_OUT = 50
_B = 16384

_NFLAT = _B * _WIN
_NW = 32
_PER_W = _NFLAT // _NW
_CB = 128
_NCHUNK = _PER_W // _CB

_sc_mesh = plsc.VectorSubcoreMesh(core_axis_name="c", subcore_axis_name="s")


_TBLK = 6400


def _trepack_body(x_ref, o_ref):
    c = x_ref.shape[1]
    e_lo = jnp.concatenate(
        [jnp.eye(_EMB, dtype=jnp.float32),
         jnp.zeros((_EMB, _EMB), jnp.float32)], axis=1)
    e_hi = jnp.concatenate(
        [jnp.zeros((_EMB, _EMB), jnp.float32),
         jnp.eye(_EMB, dtype=jnp.float32)], axis=1)
    x = x_ref[...]
    o_ref[...] = (
        lax.dot_general(x[:, :c // 2], e_lo, (((0,), (0,)), ((), ())),
                        preferred_element_type=jnp.float32)
        + lax.dot_general(x[:, c // 2:], e_hi, (((0,), (0,)), ((), ())),
                          preferred_element_type=jnp.float32))


def _make_trepack(rows):
    grid = (rows + _TBLK - 1) // _TBLK
    return pl.pallas_call(
        _trepack_body,
        grid=(grid,),
        in_specs=[pl.BlockSpec((_EMB, _TBLK), lambda i: (0, i))],
        out_specs=pl.BlockSpec((_TBLK // 2, 2 * _EMB), lambda i: (i, 0)),
        out_shape=jax.ShapeDtypeStruct((grid * (_TBLK // 2), 2 * _EMB),
                                       jnp.float32),
    )


_trepack_word = _make_trepack(1000000)
_trepack_affix = _make_trepack(100000)


def _pair_index(r):
    half = _TBLK // 2
    return (r // _TBLK) * half + r % half, (r % _TBLK) // half


def _make_sc_gather(name):
    @functools.partial(
        pl.kernel,
        mesh=_sc_mesh,
        name=name,
        compiler_params=pltpu.CompilerParams(needs_layout_passes=False),
        out_type=jax.ShapeDtypeStruct((_WIN * _EMB, _B), jnp.float32),
        scratch_types=[
            pltpu.VMEM((_PER_W,), jnp.int32),
            pltpu.VMEM((_PER_W,), jnp.int32),
            pltpu.VMEM((2, _CB, 2 * _EMB), jnp.float32),
            pltpu.VMEM((_EMB, _CB), jnp.float32),
            pltpu.SemaphoreType.DMA,
            pltpu.SemaphoreType.DMA,
        ],
    )
    def gather(table, qidx, par, out, q_v, p_v, rows, comb_t, sem0, sem1):
        wid = lax.axis_index("s") * 2 + lax.axis_index("c")
        base = wid * _PER_W
        pltpu.sync_copy(qidx.at[pl.ds(base, _PER_W)], q_v)
        pltpu.sync_copy(par.at[pl.ds(base, _PER_W)], p_v)
        sems = (sem0, sem1)

        def start(c):
            buf = c % 2
            return pltpu.async_copy(
                table.at[q_v.at[pl.ds(c * _CB, _CB)]],
                rows.at[buf], sems[buf])

        iota = lax.iota(jnp.int32, 16)
        pending = start(0)
        for c in range(_NCHUNK):
            cur = pending
            if c + 1 < _NCHUNK:
                pending = start(c + 1)
            cur.wait()
            buf = c % 2

            def grp_body(g, _):
                rows16 = g * 16 + iota
                cols = p_v[pl.ds(c * _CB + g * 16, 16)] * _EMB

                def col_body(e, _):
                    comb_t[e, pl.ds(g * 16, 16)] = plsc.load_gather(
                        rows.at[buf], [rows16, cols + e])
                    return 0

                lax.fori_loop(0, _EMB, col_body, 0, unroll=8)
                return 0

            lax.fori_loop(0, _CB // 16, grp_body, 0)
            flat = base + c * _CB
            pltpu.sync_copy(
                comb_t,
                out.at[pl.ds((flat // _B) * _EMB, _EMB),
                       pl.ds(flat % _B, _CB)])

    return gather


_gather_w = _make_sc_gather("sc_gather_word")
_gather_p = _make_sc_gather("sc_gather_prefix")
_gather_s = _make_sc_gather("sc_gather_suffix")


def _mlp_body(xw_ref, xp_ref, xs_ref, w1_ref, b1_ref, w2_ref, b2_ref,
              ot_ref):
    x = xw_ref[...] + xp_ref[...] + xs_ref[...]
    ht = jnp.tanh(
        lax.dot_general(w1_ref[...], x, (((0,), (0,)), ((), ())),
                        preferred_element_type=jnp.float32)
        + b1_ref[...])
    ot_ref[...] = (
        lax.dot_general(w2_ref[...], ht, (((0,), (0,)), ((), ())),
                        preferred_element_type=jnp.float32)
        + b2_ref[...])


_BM = 2048
_NB = _B // _BM

_mlp = pl.pallas_call(
    _mlp_body,
    grid=(_NB,),
    in_specs=[
        pl.BlockSpec((_WIN * _EMB, _BM), lambda i: (0, i)),
        pl.BlockSpec((_WIN * _EMB, _BM), lambda i: (0, i)),
        pl.BlockSpec((_WIN * _EMB, _BM), lambda i: (0, i)),
        pl.BlockSpec((_WIN * _EMB, _HID), lambda i: (0, 0)),
        pl.BlockSpec((_HID, 1), lambda i: (0, 0)),
        pl.BlockSpec((_HID, _OUT), lambda i: (0, 0)),
        pl.BlockSpec((_OUT, 1), lambda i: (0, 0)),
    ],
    out_specs=pl.BlockSpec((_OUT, _BM), lambda i: (0, i)),
    out_shape=jax.ShapeDtypeStruct((_OUT, _B), jnp.float32),
)


def kernel(words, prefixes, suffixes, word_emb, prefix_emb, suffix_emb,
           W1, b1, W2, b2):
    p2 = _trepack_affix(prefix_emb.T)
    s2 = _trepack_affix(suffix_emb.T)
    w2 = _trepack_word(word_emb.T)
    iw = words.T.reshape(-1)
    ip = prefixes.T.reshape(-1)
    is_ = suffixes.T.reshape(-1)
    qw, hw = _pair_index(iw)
    qp, hp = _pair_index(ip)
    qs, hs = _pair_index(is_)
    xp = _gather_p(p2, qp, hp)
    xs = _gather_s(s2, qs, hs)
    xw = _gather_w(w2, qw, hw)
    ot = _mlp(xw, xp, xs, W1, b1.reshape(_HID, 1), W2,
              b2.reshape(_OUT, 1))
    return ot.T


# speedup vs baseline: 2.0841x; 1.3867x over previous
import functools

import jax
import jax.numpy as jnp
from jax import lax
from jax.experimental import pallas as pl
from jax.experimental.pallas import tpu as pltpu
from jax.experimental.pallas import tpu_sc as plsc

_EMB = 64
_WIN = 5
_HID = 512
_OUT = 50
_B = 16384

_NFLAT = _B * _WIN
_NW = 32
_PER_W = _NFLAT // _NW
_CB = 256
_NCHUNK = _PER_W // _CB

_sc_mesh = plsc.VectorSubcoreMesh(core_axis_name="c", subcore_axis_name="s")


_TBLK = 6400


def _trepack_body(x_ref, o_ref):
    c = x_ref.shape[1]
    e_lo = jnp.concatenate(
        [jnp.eye(_EMB, dtype=jnp.float32),
         jnp.zeros((_EMB, _EMB), jnp.float32)], axis=1)
    e_hi = jnp.concatenate(
        [jnp.zeros((_EMB, _EMB), jnp.float32),
         jnp.eye(_EMB, dtype=jnp.float32)], axis=1)
    x = x_ref[...]
    o_ref[...] = (
        lax.dot_general(x[:, :c // 2], e_lo, (((0,), (0,)), ((), ())),
                        preferred_element_type=jnp.float32)
        + lax.dot_general(x[:, c // 2:], e_hi, (((0,), (0,)), ((), ())),
                          preferred_element_type=jnp.float32))


def _make_trepack(rows):
    grid = (rows + _TBLK - 1) // _TBLK
    return pl.pallas_call(
        _trepack_body,
        grid=(grid,),
        in_specs=[pl.BlockSpec((_EMB, _TBLK), lambda i: (0, i))],
        out_specs=pl.BlockSpec((_TBLK // 2, 2 * _EMB), lambda i: (i, 0)),
        out_shape=jax.ShapeDtypeStruct((grid * (_TBLK // 2), 2 * _EMB),
                                       jnp.float32),
    )


_trepack_word = _make_trepack(1000000)
_trepack_affix = _make_trepack(100000)


def _pair_index(r):
    half = _TBLK // 2
    return (r // _TBLK) * half + r % half, (r % _TBLK) // half


def _make_sc_gather(name):
    @functools.partial(
        pl.kernel,
        mesh=_sc_mesh,
        name=name,
        compiler_params=pltpu.CompilerParams(needs_layout_passes=False),
        out_type=jax.ShapeDtypeStruct((_WIN * _EMB, _B), jnp.float32),
        scratch_types=[
            pltpu.VMEM((_PER_W,), jnp.int32),
            pltpu.VMEM((_PER_W,), jnp.int32),
            pltpu.VMEM((2, _CB, 2 * _EMB), jnp.float32),
            pltpu.VMEM((2, _EMB, _CB), jnp.float32),
            pltpu.SemaphoreType.DMA,
            pltpu.SemaphoreType.DMA,
            pltpu.SemaphoreType.DMA,
            pltpu.SemaphoreType.DMA,
        ],
    )
    def gather(table, qidx, par, out, q_v, p_v, rows, comb_t,
               sem0, sem1, osem0, osem1):
        wid = lax.axis_index("s") * 2 + lax.axis_index("c")
        base = wid * _PER_W
        pltpu.sync_copy(qidx.at[pl.ds(base, _PER_W)], q_v)
        pltpu.sync_copy(par.at[pl.ds(base, _PER_W)], p_v)
        sems = (sem0, sem1)

        def start(c):
            buf = c % 2
            return pltpu.async_copy(
                table.at[q_v.at[pl.ds(c * _CB, _CB)]],
                rows.at[buf], sems[buf])

        iota = lax.iota(jnp.int32, 16)
        osems = (osem0, osem1)
        pending = start(0)
        owrites = [None, None]
        for c in range(_NCHUNK):
            cur = pending
            if c + 1 < _NCHUNK:
                pending = start(c + 1)
            cur.wait()
            buf = c % 2
            if owrites[buf] is not None:
                owrites[buf].wait()

            def grp_body(g, _):
                rows16 = g * 16 + iota
                cols = p_v[pl.ds(c * _CB + g * 16, 16)] * _EMB

                def col_body(e, _):
                    comb_t[buf, e, pl.ds(g * 16, 16)] = plsc.load_gather(
                        rows.at[buf], [rows16, cols + e])
                    return 0

                lax.fori_loop(0, _EMB, col_body, 0, unroll=8)
                return 0

            lax.fori_loop(0, _CB // 16, grp_body, 0)
            flat = base + c * _CB
            owrites[buf] = pltpu.async_copy(
                comb_t.at[buf],
                out.at[pl.ds((flat // _B) * _EMB, _EMB),
                       pl.ds(flat % _B, _CB)],
                osems[buf])
        for ow in owrites:
            if ow is not None:
                ow.wait()

    return gather


_gather_w = _make_sc_gather("sc_gather_word")
_gather_p = _make_sc_gather("sc_gather_prefix")
_gather_s = _make_sc_gather("sc_gather_suffix")


def _mlp_body(xw_ref, xp_ref, xs_ref, w1_ref, b1_ref, w2_ref, b2_ref,
              ot_ref):
    x = xw_ref[...] + xp_ref[...] + xs_ref[...]
    ht = jnp.tanh(
        lax.dot_general(w1_ref[...], x, (((0,), (0,)), ((), ())),
                        preferred_element_type=jnp.float32)
        + b1_ref[...])
    ot_ref[...] = (
        lax.dot_general(w2_ref[...], ht, (((0,), (0,)), ((), ())),
                        preferred_element_type=jnp.float32)
        + b2_ref[...])


_BM = 2048
_NB = _B // _BM

_mlp = pl.pallas_call(
    _mlp_body,
    grid=(_NB,),
    in_specs=[
        pl.BlockSpec((_WIN * _EMB, _BM), lambda i: (0, i)),
        pl.BlockSpec((_WIN * _EMB, _BM), lambda i: (0, i)),
        pl.BlockSpec((_WIN * _EMB, _BM), lambda i: (0, i)),
        pl.BlockSpec((_WIN * _EMB, _HID), lambda i: (0, 0)),
        pl.BlockSpec((_HID, 1), lambda i: (0, 0)),
        pl.BlockSpec((_HID, _OUT), lambda i: (0, 0)),
        pl.BlockSpec((_OUT, 1), lambda i: (0, 0)),
    ],
    out_specs=pl.BlockSpec((_OUT, _BM), lambda i: (0, i)),
    out_shape=jax.ShapeDtypeStruct((_OUT, _B), jnp.float32),
)


def kernel(words, prefixes, suffixes, word_emb, prefix_emb, suffix_emb,
           W1, b1, W2, b2):
    p2 = _trepack_affix(prefix_emb.T)
    s2 = _trepack_affix(suffix_emb.T)
    wt, p2, s2 = lax.optimization_barrier((word_emb.T, p2, s2))
    w2 = _trepack_word(wt)
    iw = words.T.reshape(-1)
    ip = prefixes.T.reshape(-1)
    is_ = suffixes.T.reshape(-1)
    qw, hw = _pair_index(iw)
    qp, hp = _pair_index(ip)
    qs, hs = _pair_index(is_)
    xp = _gather_p(p2, qp, hp)
    xs = _gather_s(s2, qs, hs)
    xw = _gather_w(w2, qw, hw)
    ot = _mlp(xw, xp, xs, W1, b1.reshape(_HID, 1), W2,
              b2.reshape(_OUT, 1))
    return ot.T


# speedup vs baseline: 2.1644x; 1.0385x over previous
import functools

import jax
import jax.numpy as jnp
from jax import lax
from jax.experimental import pallas as pl
from jax.experimental.pallas import tpu as pltpu
from jax.experimental.pallas import tpu_sc as plsc

_EMB = 64
_WIN = 5
_HID = 512
_OUT = 50
_B = 16384

_NFLAT = _B * _WIN
_NW = 32
_PER_W = _NFLAT // _NW
_CB = 256
_NCHUNK = _PER_W // _CB

_sc_mesh = plsc.VectorSubcoreMesh(core_axis_name="c", subcore_axis_name="s")


_TBLK = 12800


def _trepack_body(x_ref, o_ref):
    c = x_ref.shape[1]
    e_lo = jnp.concatenate(
        [jnp.eye(_EMB, dtype=jnp.float32),
         jnp.zeros((_EMB, _EMB), jnp.float32)], axis=1)
    e_hi = jnp.concatenate(
        [jnp.zeros((_EMB, _EMB), jnp.float32),
         jnp.eye(_EMB, dtype=jnp.float32)], axis=1)
    x = x_ref[...]
    o_ref[...] = (
        lax.dot_general(x[:, :c // 2], e_lo, (((0,), (0,)), ((), ())),
                        preferred_element_type=jnp.float32)
        + lax.dot_general(x[:, c // 2:], e_hi, (((0,), (0,)), ((), ())),
                          preferred_element_type=jnp.float32))


def _make_trepack(rows):
    grid = (rows + _TBLK - 1) // _TBLK
    return pl.pallas_call(
        _trepack_body,
        grid=(grid,),
        in_specs=[pl.BlockSpec((_EMB, _TBLK), lambda i: (0, i))],
        out_specs=pl.BlockSpec((_TBLK // 2, 2 * _EMB), lambda i: (i, 0)),
        out_shape=jax.ShapeDtypeStruct((grid * (_TBLK // 2), 2 * _EMB),
                                       jnp.float32),
    )


_trepack_word = _make_trepack(1000000)
_trepack_affix = _make_trepack(100000)


def _pair_index(r):
    half = _TBLK // 2
    return (r // _TBLK) * half + r % half, (r % _TBLK) // half


def _make_sc_gather(name):
    @functools.partial(
        pl.kernel,
        mesh=_sc_mesh,
        name=name,
        compiler_params=pltpu.CompilerParams(needs_layout_passes=False),
        out_type=jax.ShapeDtypeStruct((_WIN * _EMB, _B), jnp.float32),
        scratch_types=[
            pltpu.VMEM((_PER_W,), jnp.int32),
            pltpu.VMEM((_PER_W,), jnp.int32),
            pltpu.VMEM((2, _CB, 2 * _EMB), jnp.float32),
            pltpu.VMEM((2, _EMB, _CB), jnp.float32),
            pltpu.SemaphoreType.DMA,
            pltpu.SemaphoreType.DMA,
            pltpu.SemaphoreType.DMA,
            pltpu.SemaphoreType.DMA,
        ],
    )
    def gather(table, qidx, par, out, q_v, p_v, rows, comb_t,
               sem0, sem1, osem0, osem1):
        wid = lax.axis_index("s") * 2 + lax.axis_index("c")
        base = wid * _PER_W
        pltpu.sync_copy(qidx.at[pl.ds(base, _PER_W)], q_v)
        pltpu.sync_copy(par.at[pl.ds(base, _PER_W)], p_v)
        sems = (sem0, sem1)

        def start(c):
            buf = c % 2
            return pltpu.async_copy(
                table.at[q_v.at[pl.ds(c * _CB, _CB)]],
                rows.at[buf], sems[buf])

        iota = lax.iota(jnp.int32, 16)
        osems = (osem0, osem1)
        pending = start(0)
        owrites = [None, None]
        for c in range(_NCHUNK):
            cur = pending
            if c + 1 < _NCHUNK:
                pending = start(c + 1)
            cur.wait()
            buf = c % 2
            if owrites[buf] is not None:
                owrites[buf].wait()

            def grp_body(g, _):
                rows16 = g * 16 + iota
                cols = p_v[pl.ds(c * _CB + g * 16, 16)] * _EMB

                def col_body(e, _):
                    comb_t[buf, e, pl.ds(g * 16, 16)] = plsc.load_gather(
                        rows.at[buf], [rows16, cols + e])
                    return 0

                lax.fori_loop(0, _EMB, col_body, 0, unroll=8)
                return 0

            lax.fori_loop(0, _CB // 16, grp_body, 0)
            flat = base + c * _CB
            owrites[buf] = pltpu.async_copy(
                comb_t.at[buf],
                out.at[pl.ds((flat // _B) * _EMB, _EMB),
                       pl.ds(flat % _B, _CB)],
                osems[buf])
        for ow in owrites:
            if ow is not None:
                ow.wait()

    return gather


_gather_w = _make_sc_gather("sc_gather_word")
_gather_p = _make_sc_gather("sc_gather_prefix")
_gather_s = _make_sc_gather("sc_gather_suffix")


def _mlp_body(xw_ref, xp_ref, xs_ref, w1_ref, b1_ref, w2_ref, b2_ref,
              ot_ref):
    x = xw_ref[...] + xp_ref[...] + xs_ref[...]
    ht = jnp.tanh(
        lax.dot_general(w1_ref[...], x, (((0,), (0,)), ((), ())),
                        preferred_element_type=jnp.float32)
        + b1_ref[...])
    ot_ref[...] = (
        lax.dot_general(w2_ref[...], ht, (((0,), (0,)), ((), ())),
                        preferred_element_type=jnp.float32)
        + b2_ref[...])


_BM = 2048
_NB = _B // _BM

_mlp = pl.pallas_call(
    _mlp_body,
    grid=(_NB,),
    in_specs=[
        pl.BlockSpec((_WIN * _EMB, _BM), lambda i: (0, i)),
        pl.BlockSpec((_WIN * _EMB, _BM), lambda i: (0, i)),
        pl.BlockSpec((_WIN * _EMB, _BM), lambda i: (0, i)),
        pl.BlockSpec((_WIN * _EMB, _HID), lambda i: (0, 0)),
        pl.BlockSpec((_HID, 1), lambda i: (0, 0)),
        pl.BlockSpec((_HID, _OUT), lambda i: (0, 0)),
        pl.BlockSpec((_OUT, 1), lambda i: (0, 0)),
    ],
    out_specs=pl.BlockSpec((_OUT, _BM), lambda i: (0, i)),
    out_shape=jax.ShapeDtypeStruct((_OUT, _B), jnp.float32),
)


def kernel(words, prefixes, suffixes, word_emb, prefix_emb, suffix_emb,
           W1, b1, W2, b2):
    p2 = _trepack_affix(prefix_emb.T)
    s2 = _trepack_affix(suffix_emb.T)
    wt, p2, s2 = lax.optimization_barrier((word_emb.T, p2, s2))
    w2 = _trepack_word(wt)
    iw = words.T.reshape(-1)
    ip = prefixes.T.reshape(-1)
    is_ = suffixes.T.reshape(-1)
    qw, hw = _pair_index(iw)
    qp, hp = _pair_index(ip)
    qs, hs = _pair_index(is_)
    xp = _gather_p(p2, qp, hp)
    xs = _gather_s(s2, qs, hs)
    xw = _gather_w(w2, qw, hw)
    ot = _mlp(xw, xp, xs, W1, b1.reshape(_HID, 1), W2,
              b2.reshape(_OUT, 1))
    return ot.T
